# Initial kernel scaffold; baseline (speedup 1.0000x reference)
import jax
import jax.numpy as jnp
from jax.experimental import pallas as pl


def kernel(f_atoms_solvent, f_bonds_solvent, a2b_solvent, b2a_solvent, b2revb_solvent, atom_seg_solvent, f_atoms_solute, f_bonds_solute, a2b_solute, b2a_solute, b2revb_solute, atom_seg_solute, W_i, W_h, W_o, b_o):
    raise NotImplementedError("write your pallas kernel here")


# speedup vs baseline: 1.8581x; 1.8581x over previous
import functools

import jax
import jax.numpy as jnp
from jax import lax
from jax.experimental import pallas as pl
from jax.experimental.pallas import tpu as pltpu
from jax.experimental.pallas import tpu_sc as plsc

N = 10000
E = 320000
MAX_NB = 32
ATOM_FDIM = 133
BOND_FDIM = 147
HID = 64
DEPTH = 3
N_MOLS = 512
H2 = 2 * HID

NC = 2
NS = 16
NW = NC * NS

_SC_MESH = dict(core_axis_name="c", subcore_axis_name="s")


def _tc_init(fb_sv, fb_su, W_i, Wh2):
    BE = 2000

    def body(a, b, wi, wh, inp0_o, p0_o):
        xa = jnp.dot(a[...], wi[...], preferred_element_type=jnp.float32)
        xb = jnp.dot(b[...], wi[...], preferred_element_type=jnp.float32)
        x = jnp.concatenate([xa, xb], axis=1)
        inp0_o[...] = x
        p0_o[...] = jnp.dot(jnp.maximum(x, 0.0), wh[...],
                            preferred_element_type=jnp.float32)

    return pl.pallas_call(
        body,
        grid=(E // BE,),
        in_specs=[
            pl.BlockSpec((BE, BOND_FDIM), lambda i: (i, 0)),
            pl.BlockSpec((BE, BOND_FDIM), lambda i: (i, 0)),
            pl.BlockSpec((BOND_FDIM, HID), lambda i: (0, 0)),
            pl.BlockSpec((H2, H2), lambda i: (0, 0)),
        ],
        out_specs=[
            pl.BlockSpec((BE, H2), lambda i: (i, 0)),
            pl.BlockSpec((BE, H2), lambda i: (i, 0)),
        ],
        out_shape=[
            jax.ShapeDtypeStruct((E, H2), jnp.float32),
            jax.ShapeDtypeStruct((E, H2), jnp.float32),
        ],
        compiler_params=pltpu.CompilerParams(
            dimension_semantics=("parallel",)),
    )(fb_sv, fb_su, W_i, Wh2)


def _tc_matmul(msg, Wh2):
    BE = 4000

    def body(m, wh, p_o):
        p_o[...] = jnp.dot(m[...], wh[...], preferred_element_type=jnp.float32)

    return pl.pallas_call(
        body,
        grid=(E // BE,),
        in_specs=[
            pl.BlockSpec((BE, H2), lambda i: (i, 0)),
            pl.BlockSpec((H2, H2), lambda i: (0, 0)),
        ],
        out_specs=pl.BlockSpec((BE, H2), lambda i: (i, 0)),
        out_shape=jax.ShapeDtypeStruct((E, H2), jnp.float32),
        compiler_params=pltpu.CompilerParams(
            dimension_semantics=("parallel",)),
    )(msg, Wh2)


def _tc_q(am, Wh2):
    def body(a, wh, q_o):
        q_o[...] = jnp.dot(a[...], wh[...], preferred_element_type=jnp.float32)

    return pl.pallas_call(
        body,
        out_shape=jax.ShapeDtypeStruct((N, H2), jnp.float32),
    )(am, Wh2)


def _tc_final(f_atoms, am, seg_col, Wo_a, Wo_m, b_o_row, half):
    BN = 2000
    nblk = N // BN
    lo = half * HID

    def body(fa, a, seg, woa, wom, bo, ah_o, mv_o, acc, cnt):
        i = pl.program_id(0)
        amh = a[:, lo:lo + HID]
        ah = jnp.maximum(
            jnp.dot(fa[...], woa[...], preferred_element_type=jnp.float32)
            + jnp.dot(amh, wom[...], preferred_element_type=jnp.float32)
            + bo[...], 0.0)
        ah_o[...] = ah

        onehot = (lax.broadcasted_iota(jnp.int32, (BN, N_MOLS), 1)
                  == seg[...]).astype(jnp.float32)

        @pl.when(i == 0)
        def _():
            acc[...] = jnp.zeros_like(acc)
            cnt[...] = jnp.zeros_like(cnt)

        dn = (((0,), (0,)), ((), ()))
        acc[...] += lax.dot_general(onehot, ah, dn,
                                    preferred_element_type=jnp.float32)
        cnt[...] += lax.dot_general(onehot, jnp.ones((BN, HID), jnp.float32),
                                    dn, preferred_element_type=jnp.float32)

        @pl.when(i == nblk - 1)
        def _():
            mv_o[...] = acc[...] / jnp.maximum(cnt[...], 1.0)

    return pl.pallas_call(
        body,
        grid=(nblk,),
        in_specs=[
            pl.BlockSpec((BN, ATOM_FDIM), lambda i: (i, 0)),
            pl.BlockSpec((BN, H2), lambda i: (i, 0)),
            pl.BlockSpec((BN, 1), lambda i: (i, 0)),
            pl.BlockSpec((ATOM_FDIM, HID), lambda i: (0, 0)),
            pl.BlockSpec((HID, HID), lambda i: (0, 0)),
            pl.BlockSpec((1, HID), lambda i: (0, 0)),
        ],
        out_specs=[
            pl.BlockSpec((BN, HID), lambda i: (i, 0)),
            pl.BlockSpec((N_MOLS, HID), lambda i: (0, 0)),
        ],
        out_shape=[
            jax.ShapeDtypeStruct((N, HID), jnp.float32),
            jax.ShapeDtypeStruct((N_MOLS, HID), jnp.float32),
        ],
        scratch_shapes=[
            pltpu.VMEM((N_MOLS, HID), jnp.float32),
            pltpu.VMEM((N_MOLS, HID), jnp.float32),
        ],
    )(f_atoms, am, seg_col, Wo_a, Wo_m, b_o_row)


_APW = 320
_GS_CA = 16
_GS_NCH = _APW // _GS_CA


def _sc_gather_sum(table, a2b_sv, a2b_su, apply_relu):
    mesh = plsc.VectorSubcoreMesh(**_SC_MESH)

    @functools.partial(
        pl.kernel,
        out_type=jax.ShapeDtypeStruct((N, H2), jnp.float32),
        mesh=mesh,
        scratch_types=[
            pltpu.VMEM((_GS_CA * MAX_NB,), jnp.int32),
            pltpu.VMEM((_GS_CA * MAX_NB, H2), jnp.float32),
            pltpu.VMEM((_GS_CA, H2), jnp.float32),
            pltpu.SemaphoreType.DMA,
        ],
    )
    def k(table_h, a2b_sv_h, a2b_su_h, out_h, idx_v, rows_v, out_v, sem):
        c = lax.axis_index("c")
        s = lax.axis_index("s")
        wid = s * NC + c
        start = wid * _APW

        def chunk(ci, _):
            base = jnp.minimum(start + ci * _GS_CA, N - _GS_CA)
            for half, a2b_h in ((0, a2b_sv_h), (1, a2b_su_h)):
                pltpu.sync_copy(
                    a2b_h.at[pl.ds(base * MAX_NB, _GS_CA * MAX_NB)], idx_v)
                pltpu.async_copy(table_h.at[idx_v], rows_v, sem).wait()
                cols = tuple(pl.ds((4 * half + j) * 16, 16) for j in range(4))

                def per_atom(ai, _):
                    def per_row(kk, accs):
                        r = ai * MAX_NB + kk
                        vs = [rows_v[r, sl] for sl in cols]
                        if apply_relu:
                            vs = [jnp.maximum(v, 0.0) for v in vs]
                        return tuple(a + v for a, v in zip(accs, vs))

                    accs = lax.fori_loop(
                        0, MAX_NB, per_row,
                        tuple(jnp.zeros((16,), jnp.float32) for _ in range(4)))
                    for sl, a in zip(cols, accs):
                        out_v[ai, sl] = a
                    return 0

                lax.fori_loop(0, _GS_CA, per_atom, 0)
            pltpu.sync_copy(out_v, out_h.at[pl.ds(base, _GS_CA)])
            return 0

        lax.fori_loop(0, _GS_NCH, chunk, 0)

    return k(table, a2b_sv, a2b_su)


_BPW = E // NW
_MU_CB = 200
_MU_NCH = _BPW // _MU_CB


def _sc_msg_update(inp0, P, Q, b2a_sv, b2revb_sv, b2a_su, b2revb_su):
    mesh = plsc.VectorSubcoreMesh(**_SC_MESH)

    @functools.partial(
        pl.kernel,
        out_type=jax.ShapeDtypeStruct((E, H2), jnp.float32),
        mesh=mesh,
        scratch_types=[
            pltpu.VMEM((_MU_CB,), jnp.int32),
            pltpu.VMEM((_MU_CB,), jnp.int32),
            pltpu.VMEM((_MU_CB, H2), jnp.float32),
            pltpu.VMEM((_MU_CB, H2), jnp.float32),
            pltpu.VMEM((_MU_CB, H2), jnp.float32),
            pltpu.SemaphoreType.DMA,
        ],
    )
    def k(inp0_h, p_h, q_h, b2a_sv_h, b2revb_sv_h, b2a_su_h, b2revb_su_h,
          out_h, ia_v, ir_v, q_v, p_v, x_v, sem):
        c = lax.axis_index("c")
        s = lax.axis_index("s")
        wid = s * NC + c
        start = wid * _BPW

        def chunk(ci, _):
            base = start + ci * _MU_CB
            cx = pltpu.async_copy(inp0_h.at[pl.ds(base, _MU_CB)], x_v, sem)
            cx.wait()
            for half, b2a_h, b2revb_h in ((0, b2a_sv_h, b2revb_sv_h),
                                          (1, b2a_su_h, b2revb_su_h)):
                pltpu.sync_copy(b2a_h.at[pl.ds(base, _MU_CB)], ia_v)
                pltpu.sync_copy(b2revb_h.at[pl.ds(base, _MU_CB)], ir_v)
                cq = pltpu.async_copy(q_h.at[ia_v], q_v, sem)
                cp = pltpu.async_copy(p_h.at[ir_v], p_v, sem)
                cq.wait()
                cp.wait()

                def per_row(r, _):
                    for j in range(4):
                        sl = pl.ds((4 * half + j) * 16, 16)
                        x_v[r, sl] = jnp.maximum(
                            x_v[r, sl] + q_v[r, sl] - p_v[r, sl], 0.0)
                    return 0

                lax.fori_loop(0, _MU_CB, per_row, 0)
            pltpu.sync_copy(x_v, out_h.at[pl.ds(base, _MU_CB)])
            return 0

        lax.fori_loop(0, _MU_NCH, chunk, 0)

    return k(inp0, P, Q, b2a_sv, b2revb_sv, b2a_su, b2revb_su)


def kernel(f_atoms_solvent, f_bonds_solvent, a2b_solvent, b2a_solvent,
           b2revb_solvent, atom_seg_solvent, f_atoms_solute, f_bonds_solute,
           a2b_solute, b2a_solute, b2revb_solute, atom_seg_solute,
           W_i, W_h, W_o, b_o):
    a2b_sv = a2b_solvent.reshape(-1).astype(jnp.int32)
    a2b_su = a2b_solute.reshape(-1).astype(jnp.int32)
    b2a_sv = b2a_solvent.astype(jnp.int32)
    b2a_su = b2a_solute.astype(jnp.int32)
    b2revb_sv = b2revb_solvent.astype(jnp.int32)
    b2revb_su = b2revb_solute.astype(jnp.int32)
    seg_sv = atom_seg_solvent.astype(jnp.int32).reshape(N, 1)
    seg_su = atom_seg_solute.astype(jnp.int32).reshape(N, 1)

    Wh2 = jnp.zeros((H2, H2), jnp.float32)
    Wh2 = Wh2.at[:HID, :HID].set(W_h).at[HID:, HID:].set(W_h)
    Wo_a = W_o[:ATOM_FDIM]
    Wo_m = W_o[ATOM_FDIM:]
    b_o_row = b_o.reshape(1, HID)

    inp0, P = _tc_init(f_bonds_solvent, f_bonds_solute, W_i, Wh2)
    msg = inp0
    for t in range(DEPTH - 1):
        am = _sc_gather_sum(msg, a2b_sv, a2b_su, apply_relu=(t == 0))
        Q = _tc_q(am, Wh2)
        msg = _sc_msg_update(inp0, P, Q, b2a_sv, b2revb_sv, b2a_su, b2revb_su)
        if t < DEPTH - 2:
            P = _tc_matmul(msg, Wh2)
    am_f = _sc_gather_sum(msg, a2b_sv, a2b_su, apply_relu=False)
    ah_sv, mv_sv = _tc_final(f_atoms_solvent, am_f, seg_sv, Wo_a, Wo_m,
                             b_o_row, half=0)
    ah_su, mv_su = _tc_final(f_atoms_solute, am_f, seg_su, Wo_a, Wo_m,
                             b_o_row, half=1)
    return (mv_sv, mv_su, ah_sv, ah_su)


# speedup vs baseline: 2.1918x; 1.1796x over previous
import functools

import jax
import jax.numpy as jnp
from jax import lax
from jax.experimental import pallas as pl
from jax.experimental.pallas import tpu as pltpu
from jax.experimental.pallas import tpu_sc as plsc

N = 10000
E = 320000
MAX_NB = 32
ATOM_FDIM = 133
BOND_FDIM = 147
HID = 64
DEPTH = 3
N_MOLS = 512
H2 = 2 * HID

NC = 2
NS = 16
NW = NC * NS

_SC_MESH = dict(core_axis_name="c", subcore_axis_name="s")


def _tc_init(fb_sv, fb_su, W_i, Wh2):
    BE = 2000

    def body(a, b, wi, wh, inp0_o, p0_o):
        xa = jnp.dot(a[...], wi[...], preferred_element_type=jnp.float32)
        xb = jnp.dot(b[...], wi[...], preferred_element_type=jnp.float32)
        x = jnp.concatenate([xa, xb], axis=1)
        inp0_o[...] = x
        p0_o[...] = jnp.dot(jnp.maximum(x, 0.0), wh[...],
                            preferred_element_type=jnp.float32)

    return pl.pallas_call(
        body,
        grid=(E // BE,),
        in_specs=[
            pl.BlockSpec((BE, BOND_FDIM), lambda i: (i, 0)),
            pl.BlockSpec((BE, BOND_FDIM), lambda i: (i, 0)),
            pl.BlockSpec((BOND_FDIM, HID), lambda i: (0, 0)),
            pl.BlockSpec((H2, H2), lambda i: (0, 0)),
        ],
        out_specs=[
            pl.BlockSpec((BE, H2), lambda i: (i, 0)),
            pl.BlockSpec((BE, H2), lambda i: (i, 0)),
        ],
        out_shape=[
            jax.ShapeDtypeStruct((E, H2), jnp.float32),
            jax.ShapeDtypeStruct((E, H2), jnp.float32),
        ],
        compiler_params=pltpu.CompilerParams(
            dimension_semantics=("parallel",)),
    )(fb_sv, fb_su, W_i, Wh2)


def _tc_matmul(msg, Wh2):
    BE = 4000

    def body(m, wh, p_o):
        p_o[...] = jnp.dot(m[...], wh[...], preferred_element_type=jnp.float32)

    return pl.pallas_call(
        body,
        grid=(E // BE,),
        in_specs=[
            pl.BlockSpec((BE, H2), lambda i: (i, 0)),
            pl.BlockSpec((H2, H2), lambda i: (0, 0)),
        ],
        out_specs=pl.BlockSpec((BE, H2), lambda i: (i, 0)),
        out_shape=jax.ShapeDtypeStruct((E, H2), jnp.float32),
        compiler_params=pltpu.CompilerParams(
            dimension_semantics=("parallel",)),
    )(msg, Wh2)


def _tc_q(am, Wh2):
    def body(a, wh, q_o):
        q_o[...] = jnp.dot(a[...], wh[...], preferred_element_type=jnp.float32)

    return pl.pallas_call(
        body,
        out_shape=jax.ShapeDtypeStruct((N, H2), jnp.float32),
    )(am, Wh2)


def _tc_final(f_atoms, am, seg_col, Wo_a, Wo_m, b_o_row, half):
    BN = 2000
    nblk = N // BN
    lo = half * HID

    def body(fa, a, seg, woa, wom, bo, ah_o, mv_o, acc, cnt):
        i = pl.program_id(0)
        amh = a[:, lo:lo + HID]
        ah = jnp.maximum(
            jnp.dot(fa[...], woa[...], preferred_element_type=jnp.float32)
            + jnp.dot(amh, wom[...], preferred_element_type=jnp.float32)
            + bo[...], 0.0)
        ah_o[...] = ah

        onehot = (lax.broadcasted_iota(jnp.int32, (BN, N_MOLS), 1)
                  == seg[...]).astype(jnp.float32)

        @pl.when(i == 0)
        def _():
            acc[...] = jnp.zeros_like(acc)
            cnt[...] = jnp.zeros_like(cnt)

        dn = (((0,), (0,)), ((), ()))
        acc[...] += lax.dot_general(onehot, ah, dn,
                                    preferred_element_type=jnp.float32)
        cnt[...] += lax.dot_general(onehot, jnp.ones((BN, HID), jnp.float32),
                                    dn, preferred_element_type=jnp.float32)

        @pl.when(i == nblk - 1)
        def _():
            mv_o[...] = acc[...] / jnp.maximum(cnt[...], 1.0)

    return pl.pallas_call(
        body,
        grid=(nblk,),
        in_specs=[
            pl.BlockSpec((BN, ATOM_FDIM), lambda i: (i, 0)),
            pl.BlockSpec((BN, H2), lambda i: (i, 0)),
            pl.BlockSpec((BN, 1), lambda i: (i, 0)),
            pl.BlockSpec((ATOM_FDIM, HID), lambda i: (0, 0)),
            pl.BlockSpec((HID, HID), lambda i: (0, 0)),
            pl.BlockSpec((1, HID), lambda i: (0, 0)),
        ],
        out_specs=[
            pl.BlockSpec((BN, HID), lambda i: (i, 0)),
            pl.BlockSpec((N_MOLS, HID), lambda i: (0, 0)),
        ],
        out_shape=[
            jax.ShapeDtypeStruct((N, HID), jnp.float32),
            jax.ShapeDtypeStruct((N_MOLS, HID), jnp.float32),
        ],
        scratch_shapes=[
            pltpu.VMEM((N_MOLS, HID), jnp.float32),
            pltpu.VMEM((N_MOLS, HID), jnp.float32),
        ],
    )(f_atoms, am, seg_col, Wo_a, Wo_m, b_o_row)


_APW = 320
_GS_CA = 8
_GS_NCH = _APW // _GS_CA
_GS_R = _GS_CA * MAX_NB


def _sc_gather_sum(table, a2b_sv, a2b_su, apply_relu):
    mesh = plsc.VectorSubcoreMesh(**_SC_MESH)

    @functools.partial(
        pl.kernel,
        out_type=jax.ShapeDtypeStruct((N, H2), jnp.float32),
        mesh=mesh,
        scratch_types=[
            pltpu.VMEM((_GS_R,), jnp.int32),
            pltpu.VMEM((_GS_R,), jnp.int32),
            pltpu.VMEM((_GS_R, H2), jnp.float32),
            pltpu.VMEM((_GS_R, H2), jnp.float32),
            pltpu.VMEM((_GS_CA, H2), jnp.float32),
            pltpu.SemaphoreType.DMA,
            pltpu.SemaphoreType.DMA,
        ],
    )
    def k(table_h, a2b_sv_h, a2b_su_h, out_h,
          idx0_v, idx1_v, rows0_v, rows1_v, out_v, sem0, sem1):
        c = lax.axis_index("c")
        s = lax.axis_index("s")
        wid = s * NC + c
        start = wid * _APW
        idx_v = (idx0_v, idx1_v)
        rows_v = (rows0_v, rows1_v)
        sems = (sem0, sem1)
        a2b_hs = (a2b_sv_h, a2b_su_h)

        def chunk_base(ci):
            return jnp.minimum(start + ci * _GS_CA, N - _GS_CA)

        def issue(ci, half):
            b = chunk_base(ci)
            pltpu.sync_copy(
                a2b_hs[half].at[pl.ds(b * MAX_NB, _GS_R)], idx_v[half])
            pltpu.async_copy(table_h.at[idx_v[half]], rows_v[half],
                             sems[half])

        def reduce_half(half):
            cols = tuple(pl.ds((4 * half + j) * 16, 16) for j in range(4))

            def per_atom(ai, _):
                def per_row(kk, accs):
                    r = ai * MAX_NB + kk
                    vs = [rows_v[half][r, sl] for sl in cols]
                    if apply_relu:
                        vs = [jnp.maximum(v, 0.0) for v in vs]
                    return tuple(a + v for a, v in zip(accs, vs))

                accs = lax.fori_loop(
                    0, MAX_NB, per_row,
                    tuple(jnp.zeros((16,), jnp.float32) for _ in range(4)))
                for sl, a in zip(cols, accs):
                    out_v[ai, sl] = a
                return 0

            lax.fori_loop(0, _GS_CA, per_atom, 0)

        def wait(half):
            pltpu.make_async_copy(table_h.at[idx_v[half]], rows_v[half],
                                  sems[half]).wait()

        issue(0, 0)

        def chunk(ci, _):
            wait(0)
            issue(ci, 1)
            reduce_half(0)
            wait(1)
            issue(lax.rem(ci + 1, _GS_NCH), 0)
            reduce_half(1)
            pltpu.sync_copy(out_v, out_h.at[pl.ds(chunk_base(ci), _GS_CA)])
            return 0

        lax.fori_loop(0, _GS_NCH, chunk, 0)
        wait(0)

    return k(table, a2b_sv, a2b_su)


_BPW = E // NW
_MU_CB = 80
_MU_NCH = _BPW // _MU_CB
_MU_NIT = (_MU_NCH + 1) // 2


def _sc_msg_update(inp0, P, Q, b2a_sv, b2revb_sv, b2a_su, b2revb_su):
    mesh = plsc.VectorSubcoreMesh(**_SC_MESH)

    @functools.partial(
        pl.kernel,
        out_type=jax.ShapeDtypeStruct((E, H2), jnp.float32),
        mesh=mesh,
        scratch_types=[
            pltpu.VMEM((_BPW,), jnp.int32),
            pltpu.VMEM((_BPW,), jnp.int32),
            pltpu.VMEM((_BPW,), jnp.int32),
            pltpu.VMEM((_BPW,), jnp.int32),
            pltpu.VMEM((_MU_CB, H2), jnp.float32),
            pltpu.VMEM((_MU_CB, H2), jnp.float32),
            pltpu.VMEM((_MU_CB, H2), jnp.float32),
            pltpu.VMEM((_MU_CB, H2), jnp.float32),
            pltpu.VMEM((_MU_CB, H2), jnp.float32),
            pltpu.VMEM((_MU_CB, H2), jnp.float32),
            pltpu.SemaphoreType.DMA,
            pltpu.SemaphoreType.DMA,
            pltpu.SemaphoreType.DMA,
            pltpu.SemaphoreType.DMA,
            pltpu.SemaphoreType.DMA,
            pltpu.SemaphoreType.DMA,
        ],
    )
    def k(inp0_h, p_h, q_h, b2a_sv_h, b2revb_sv_h, b2a_su_h,
          b2revb_su_h, out_h,
          ia0_v, ia1_v, ir0_v, ir1_v, x0_v, x1_v, p0_v, p1_v, q0_v, q1_v,
          sx0, sx1, sp0, sp1, sq0, sq1):
        c = lax.axis_index("c")
        s = lax.axis_index("s")
        wid = s * NC + c
        start = wid * _BPW

        pltpu.sync_copy(b2a_sv_h.at[pl.ds(start, _BPW)], ia0_v)
        pltpu.sync_copy(b2a_su_h.at[pl.ds(start, _BPW)], ia1_v)
        pltpu.sync_copy(b2revb_sv_h.at[pl.ds(start, _BPW)], ir0_v)
        pltpu.sync_copy(b2revb_su_h.at[pl.ds(start, _BPW)], ir1_v)

        ia = (ia0_v, ia1_v)
        ir = (ir0_v, ir1_v)
        xs = (x0_v, x1_v)
        ps = (p0_v, p1_v)
        qs = (q0_v, q1_v)
        sxs = (sx0, sx1)
        sps = (sp0, sp1)
        sqs = (sq0, sq1)

        def cl(ci):
            return jnp.minimum(ci, _MU_NCH - 1)

        def issue_x(b, ci):
            pltpu.async_copy(
                inp0_h.at[pl.ds(start + cl(ci) * _MU_CB, _MU_CB)],
                xs[b], sxs[b])

        def wait_x(b):
            pltpu.make_async_copy(inp0_h.at[pl.ds(start, _MU_CB)],
                                  xs[b], sxs[b]).wait()

        def issue_pq(b, ci, half):
            off = pl.ds(cl(ci) * _MU_CB, _MU_CB)
            pltpu.async_copy(p_h.at[ir[half].at[off]], ps[b], sps[b])
            pltpu.async_copy(q_h.at[ia[half].at[off]], qs[b], sqs[b])

        def wait_pq(b, half):
            off = pl.ds(0, _MU_CB)
            pltpu.make_async_copy(p_h.at[ir[half].at[off]],
                                  ps[b], sps[b]).wait()
            pltpu.make_async_copy(q_h.at[ia[half].at[off]],
                                  qs[b], sqs[b]).wait()

        def compute(xb, pb, half):
            def per_row(r, _):
                for j in range(4):
                    slx = pl.ds((4 * half + j) * 16, 16)
                    xs[xb][r, slx] = jnp.maximum(
                        xs[xb][r, slx] + qs[pb][r, slx] - ps[pb][r, slx], 0.0)
                return 0

            lax.fori_loop(0, _MU_CB, per_row, 0)

        def store_x(xb, ci):
            pltpu.sync_copy(xs[xb],
                            out_h.at[pl.ds(start + ci * _MU_CB, _MU_CB)])

        issue_x(0, 0)
        issue_pq(0, 0, 0)

        def iteration(i, _):
            c0 = 2 * i
            c1 = cl(2 * i + 1)
            wait_x(0)
            wait_pq(0, 0)
            issue_pq(1, c0, 1)
            compute(0, 0, 0)
            wait_pq(1, 1)
            issue_x(1, c1)
            issue_pq(0, c1, 0)
            compute(0, 1, 1)
            store_x(0, c0)
            wait_x(1)
            wait_pq(0, 0)
            issue_pq(1, c1, 1)
            compute(1, 0, 0)
            wait_pq(1, 1)
            issue_x(0, 2 * i + 2)
            issue_pq(0, 2 * i + 2, 0)
            compute(1, 1, 1)
            store_x(1, c1)
            return 0

        lax.fori_loop(0, _MU_NIT, iteration, 0)
        wait_x(0)
        wait_pq(0, 0)

    return k(inp0, P, Q, b2a_sv, b2revb_sv, b2a_su, b2revb_su)


def kernel(f_atoms_solvent, f_bonds_solvent, a2b_solvent, b2a_solvent,
           b2revb_solvent, atom_seg_solvent, f_atoms_solute, f_bonds_solute,
           a2b_solute, b2a_solute, b2revb_solute, atom_seg_solute,
           W_i, W_h, W_o, b_o):
    a2b_sv = a2b_solvent.reshape(-1).astype(jnp.int32)
    a2b_su = a2b_solute.reshape(-1).astype(jnp.int32)
    b2a_sv = b2a_solvent.astype(jnp.int32)
    b2a_su = b2a_solute.astype(jnp.int32)
    b2revb_sv = b2revb_solvent.astype(jnp.int32)
    b2revb_su = b2revb_solute.astype(jnp.int32)
    seg_sv = atom_seg_solvent.astype(jnp.int32).reshape(N, 1)
    seg_su = atom_seg_solute.astype(jnp.int32).reshape(N, 1)

    Wh2 = jnp.zeros((H2, H2), jnp.float32)
    Wh2 = Wh2.at[:HID, :HID].set(W_h).at[HID:, HID:].set(W_h)
    Wo_a = W_o[:ATOM_FDIM]
    Wo_m = W_o[ATOM_FDIM:]
    b_o_row = b_o.reshape(1, HID)

    inp0, P = _tc_init(f_bonds_solvent, f_bonds_solute, W_i, Wh2)
    msg = inp0
    for t in range(DEPTH - 1):
        am = _sc_gather_sum(msg, a2b_sv, a2b_su, apply_relu=(t == 0))
        Q = _tc_q(am, Wh2)
        msg = _sc_msg_update(inp0, P, Q,
                             b2a_sv, b2revb_sv, b2a_su, b2revb_su)
        if t < DEPTH - 2:
            P = _tc_matmul(msg, Wh2)
    am_f = _sc_gather_sum(msg, a2b_sv, a2b_su, apply_relu=False)
    ah_sv, mv_sv = _tc_final(f_atoms_solvent, am_f, seg_sv, Wo_a, Wo_m,
                             b_o_row, half=0)
    ah_su, mv_su = _tc_final(f_atoms_solute, am_f, seg_su, Wo_a, Wo_m,
                             b_o_row, half=1)
    return (mv_sv, mv_su, ah_sv, ah_su)
